# baseline (device time: 234211 ns/iter reference)
import jax
import jax.numpy as jnp
from jax import lax
from jax.experimental import pallas as pl
from jax.experimental.pallas import tpu as pltpu

M, N = 8192, 1024
HALF = M // 2
K = 8
R = HALF // K


def kernel(x):
    def body(x_hbm, out_hbm, vrecv, vx, xs_sems, xr_sems, ys_sems, yr_sems,
             ld_sems, st_sems):
        mx = lax.axis_index("x")
        my = lax.axis_index("y")
        mz = lax.axis_index("z")
        xp = (1 - mx, my, mz)
        yp = (mx, 1 - my, mz)

        barrier_sem = pltpu.get_barrier_semaphore()
        for nbr in (xp, yp):
            pl.semaphore_signal(
                barrier_sem, inc=1,
                device_id=nbr, device_id_type=pl.DeviceIdType.MESH,
            )
        pl.semaphore_wait(barrier_sem, 2)

        h0 = my * HALF

        lds = []
        for c in range(K):
            ld = pltpu.make_async_copy(
                x_hbm.at[pl.ds(h0 + c * R, R)], vx.at[c], ld_sems.at[c])
            ld.start()
            lds.append(ld)

        x_rdmas = []
        for c in range(K):
            lds[c].wait()
            r = pltpu.make_async_remote_copy(
                src_ref=vx.at[c],
                dst_ref=vrecv.at[c],
                send_sem=xs_sems.at[c],
                recv_sem=xr_sems.at[c],
                device_id=xp,
                device_id_type=pl.DeviceIdType.MESH,
            )
            r.start()
            x_rdmas.append(r)

        y_rdmas = []
        sts = []
        for c in range(K):
            x_rdmas[c].wait_recv()
            vrecv[c] = vrecv[c] + vx[c]
            yr = pltpu.make_async_remote_copy(
                src_ref=vrecv.at[c],
                dst_ref=out_hbm.at[pl.ds(h0 + c * R, R)],
                send_sem=ys_sems.at[c],
                recv_sem=yr_sems.at[c],
                device_id=yp,
                device_id_type=pl.DeviceIdType.MESH,
            )
            yr.start()
            y_rdmas.append(yr)
            st = pltpu.make_async_copy(
                vrecv.at[c], out_hbm.at[pl.ds(h0 + c * R, R)],
                st_sems.at[c])
            st.start()
            sts.append(st)

        for c in range(K):
            x_rdmas[c].wait_send()
            y_rdmas[c].wait_send()
            y_rdmas[c].wait_recv()
            sts[c].wait()

    return pl.pallas_call(
        body,
        out_shape=jax.ShapeDtypeStruct((M, N), jnp.float32),
        in_specs=[pl.BlockSpec(memory_space=pl.ANY)],
        out_specs=pl.BlockSpec(memory_space=pl.ANY),
        scratch_shapes=[
            pltpu.VMEM((K, R, N), jnp.float32),
            pltpu.VMEM((K, R, N), jnp.float32),
            pltpu.SemaphoreType.DMA((K,)),
            pltpu.SemaphoreType.DMA((K,)),
            pltpu.SemaphoreType.DMA((K,)),
            pltpu.SemaphoreType.DMA((K,)),
            pltpu.SemaphoreType.DMA((K,)),
            pltpu.SemaphoreType.DMA((K,)),
        ],
        compiler_params=pltpu.CompilerParams(collective_id=0),
    )(x)


# device time: 218023 ns/iter; 1.0742x vs baseline; 1.0742x over previous
import jax
import jax.numpy as jnp
from jax import lax
from jax.experimental import pallas as pl
from jax.experimental.pallas import tpu as pltpu

M, N = 8192, 1024
HALF = M // 2
K = 32
R = HALF // K


def kernel(x):
    def body(x_hbm, out_hbm, vrecv, vx, xs_sems, xr_sems, ys_sems, yr_sems,
             ld_sems, st_sems):
        mx = lax.axis_index("x")
        my = lax.axis_index("y")
        mz = lax.axis_index("z")
        xp = (1 - mx, my, mz)
        yp = (mx, 1 - my, mz)

        barrier_sem = pltpu.get_barrier_semaphore()
        for nbr in (xp, yp):
            pl.semaphore_signal(
                barrier_sem, inc=1,
                device_id=nbr, device_id_type=pl.DeviceIdType.MESH,
            )
        pl.semaphore_wait(barrier_sem, 2)

        h0 = my * HALF

        lds = []
        for c in range(K):
            ld = pltpu.make_async_copy(
                x_hbm.at[pl.ds(h0 + c * R, R)], vx.at[c], ld_sems.at[c])
            ld.start()
            lds.append(ld)

        x_rdmas = []
        for c in range(K):
            lds[c].wait()
            r = pltpu.make_async_remote_copy(
                src_ref=vx.at[c],
                dst_ref=vrecv.at[c],
                send_sem=xs_sems.at[c],
                recv_sem=xr_sems.at[c],
                device_id=xp,
                device_id_type=pl.DeviceIdType.MESH,
            )
            r.start()
            x_rdmas.append(r)

        y_rdmas = []
        sts = []
        for c in range(K):
            x_rdmas[c].wait_recv()
            vrecv[c] = vrecv[c] + vx[c]
            yr = pltpu.make_async_remote_copy(
                src_ref=vrecv.at[c],
                dst_ref=out_hbm.at[pl.ds(h0 + c * R, R)],
                send_sem=ys_sems.at[c],
                recv_sem=yr_sems.at[c],
                device_id=yp,
                device_id_type=pl.DeviceIdType.MESH,
            )
            yr.start()
            y_rdmas.append(yr)
            st = pltpu.make_async_copy(
                vrecv.at[c], out_hbm.at[pl.ds(h0 + c * R, R)],
                st_sems.at[c])
            st.start()
            sts.append(st)

        for c in range(K):
            x_rdmas[c].wait_send()
            y_rdmas[c].wait_send()
            y_rdmas[c].wait_recv()
            sts[c].wait()

    return pl.pallas_call(
        body,
        out_shape=jax.ShapeDtypeStruct((M, N), jnp.float32),
        in_specs=[pl.BlockSpec(memory_space=pl.ANY)],
        out_specs=pl.BlockSpec(memory_space=pl.ANY),
        scratch_shapes=[
            pltpu.VMEM((K, R, N), jnp.float32),
            pltpu.VMEM((K, R, N), jnp.float32),
            pltpu.SemaphoreType.DMA((K,)),
            pltpu.SemaphoreType.DMA((K,)),
            pltpu.SemaphoreType.DMA((K,)),
            pltpu.SemaphoreType.DMA((K,)),
            pltpu.SemaphoreType.DMA((K,)),
            pltpu.SemaphoreType.DMA((K,)),
        ],
        compiler_params=pltpu.CompilerParams(collective_id=0),
    )(x)
